# baseline (speedup 1.0000x reference)
import functools

import jax
import jax.numpy as jnp
from jax import lax
from jax.experimental import pallas as pl
from jax.experimental.pallas import tpu as pltpu
from jax.experimental.pallas import tpu_sc as plsc

B, L, V = 1024, 50, 1000
NC, NS, LANES = 2, 16, 16
NW = NC * NS
SB = 0
NB_SC = SB // NW
PER_W = NB_SC * L
BB = 32
GROUPS = (0, 16, 32, 34)


@functools.cache
def _make_sc_gather():
    mesh = plsc.VectorSubcoreMesh(core_axis_name="c", subcore_axis_name="s")
    return functools.partial(
        pl.kernel,
        mesh=mesh,
        out_type=[
            jax.ShapeDtypeStruct((SB * L,), jnp.float32),
            jax.ShapeDtypeStruct((SB * L,), jnp.float32),
        ],
        scratch_types=[
            pltpu.VMEM((PER_W,), jnp.int32),
            pltpu.VMEM((PER_W,), jnp.int32),
            pltpu.VMEM((L, V), jnp.float32),
            pltpu.VMEM((L, V), jnp.float32),
            pltpu.VMEM((PER_W,), jnp.float32),
            pltpu.VMEM((PER_W,), jnp.float32),
            pltpu.SemaphoreType.DMA,
            pltpu.SemaphoreType.DMA,
        ],
        compiler_params=pltpu.CompilerParams(
            use_tc_tiling_on_sc=True,
            disable_bounds_checks=True,
        ),
    )(_sc_gather_body)


def _sc_gather_body(scores_hbm, pos_hbm, neg_hbm, pout_hbm, nout_hbm,
                    pidx_v, nidx_v, buf0, buf1, pg_v, ng_v, sem0, sem1):
    wid = lax.axis_index("s") * NC + lax.axis_index("c")
    base = wid * PER_W
    b0 = wid * NB_SC

    pltpu.sync_copy(pos_hbm.at[pl.ds(base, PER_W)], pidx_v)
    pltpu.sync_copy(neg_hbm.at[pl.ds(base, PER_W)], nidx_v)

    def _extract(buf, bi):
        off = bi * L
        lane = lax.iota(jnp.int32, LANES)
        for g in GROUPS:
            pi16 = pidx_v[pl.ds(off + g, LANES)]
            ni16 = nidx_v[pl.ds(off + g, LANES)]
            pc0 = pi16 & ~(LANES - 1)
            nc0 = ni16 & ~(LANES - 1)
            pcm = pi16 & (LANES - 1)
            ncm = ni16 & (LANES - 1)
            pacc = jnp.zeros((LANES,), jnp.float32)
            nacc = jnp.zeros((LANES,), jnp.float32)
            for k in range(LANES):
                row = g + k
                pv = buf[row, pl.ds(pl.multiple_of(pc0[k], LANES), LANES)]
                nv = buf[row, pl.ds(pl.multiple_of(nc0[k], LANES), LANES)]
                ps = jnp.take(pv, pcm)[k]
                ns = jnp.take(nv, ncm)[k]
                pacc = jnp.where(lane == k, ps, pacc)
                nacc = jnp.where(lane == k, ns, nacc)
            pg_v[pl.ds(off + g, LANES)] = pacc
            ng_v[pl.ds(off + g, LANES)] = nacc

    pltpu.make_async_copy(scores_hbm.at[b0], buf0, sem0).start()

    def _step(bi, carry):
        nxt = bi + 1

        @pl.when(nxt < NB_SC)
        def _prefetch():
            @pl.when(nxt % 2 == 0)
            def _():
                pltpu.make_async_copy(
                    scores_hbm.at[b0 + nxt], buf0, sem0).start()

            @pl.when(nxt % 2 == 1)
            def _():
                pltpu.make_async_copy(
                    scores_hbm.at[b0 + nxt], buf1, sem1).start()

        @pl.when(bi % 2 == 0)
        def _use0():
            pltpu.make_async_copy(scores_hbm.at[b0], buf0, sem0).wait()
            _extract(buf0, bi)

        @pl.when(bi % 2 == 1)
        def _use1():
            pltpu.make_async_copy(scores_hbm.at[b0], buf1, sem1).wait()
            _extract(buf1, bi)

        return carry

    lax.fori_loop(0, NB_SC, _step, 0)

    pltpu.sync_copy(pg_v, pout_hbm.at[pl.ds(base, PER_W)])
    pltpu.sync_copy(ng_v, nout_hbm.at[pl.ds(base, PER_W)])


def _tc_extract_body(s_ref, pn_ref, w_ref, acc_ref):
    i = pl.program_id(0)
    s = s_ref[...]
    pn = pn_ref[...]
    w = w_ref[...]
    hi = pn >> 7
    lo = pn & 127
    g = jnp.zeros(pn.shape, jnp.float32)
    for j in range(8):
        c0 = 128 * j
        cl = min(128, V - c0)
        idx = jnp.minimum(lo, cl - 1)
        gj = jnp.take_along_axis(
            s[:, :, c0:c0 + cl], idx, axis=2, mode="promise_in_bounds")
        g = g + jnp.where(hi == j, gj, 0.0)
    x = (g[..., 0] - g[..., 1]) * w
    ls = jnp.minimum(x, 0.0) - jnp.log1p(jnp.exp(-jnp.abs(x)))
    v = jnp.logical_and(pn[..., 0] > 0, pn[..., 1] > 0).astype(jnp.float32)

    @pl.when(i == 0)
    def _():
        acc_ref[...] = jnp.zeros((1, 2), jnp.float32)

    part = jnp.concatenate(
        [jnp.sum(ls * v).reshape(1, 1), jnp.sum(v).reshape(1, 1)], axis=1)
    acc_ref[...] += part


def _tc_extract(scores, pos, neg, w):
    nb = B - SB
    grid = (nb // BB,)
    pn = jnp.stack([pos, neg], axis=-1)
    off = SB // BB
    return pl.pallas_call(
        _tc_extract_body,
        grid=grid,
        in_specs=[
            pl.BlockSpec((BB, L, V), lambda i: (off + i, 0, 0)),
            pl.BlockSpec((BB, L, 2), lambda i: (off + i, 0, 0)),
            pl.BlockSpec((BB, L), lambda i: (off + i, 0)),
        ],
        out_specs=pl.BlockSpec((1, 2), lambda i: (0, 0)),
        out_shape=jax.ShapeDtypeStruct((1, 2), jnp.float32),
    )(scores, pn, w)


def _sc_loss_body(pg_ref, ng_ref, w_ref, pi_ref, ni_ref, out_ref):
    pg = pg_ref[...]
    ng = ng_ref[...]
    w = w_ref[...]
    valid = jnp.logical_and(pi_ref[...] > 0, ni_ref[...] > 0)
    x = (pg - ng) * w
    ls = jnp.minimum(x, 0.0) - jnp.log1p(jnp.exp(-jnp.abs(x)))
    v = valid.astype(jnp.float32)
    out_ref[...] = jnp.concatenate(
        [jnp.sum(ls * v).reshape(1, 1), jnp.sum(v).reshape(1, 1)], axis=1)


def kernel(scores, positive_mask, negative_mask, weights):
    pos = positive_mask.astype(jnp.int32)
    neg = negative_mask.astype(jnp.int32)
    acc_tc = _tc_extract(scores, pos, neg, weights)
    if SB > 0:
        nsc = SB * L
        psl = pos[:SB].reshape(-1)
        nsl = neg[:SB].reshape(-1)
        pg, ng = _make_sc_gather()(scores, psl, nsl)
        Rs, Cs = nsc // 128, 128
        acc_sc = pl.pallas_call(
            _sc_loss_body,
            out_shape=jax.ShapeDtypeStruct((1, 2), jnp.float32),
        )(pg.reshape(Rs, Cs), ng.reshape(Rs, Cs),
          weights[:SB].reshape(Rs, Cs),
          psl.reshape(Rs, Cs), nsl.reshape(Rs, Cs))
        acc = acc_tc + acc_sc
    else:
        acc = acc_tc
    s, c = acc[0, 0], acc[0, 1]
    return -jnp.where(c == 0.0, jnp.float32(0.0), s / jnp.maximum(c, 1.0))


# speedup vs baseline: 1.4803x; 1.4803x over previous
import functools

import jax
import jax.numpy as jnp
from jax import lax
from jax.experimental import pallas as pl
from jax.experimental.pallas import tpu as pltpu
from jax.experimental.pallas import tpu_sc as plsc

B, L, V = 1024, 50, 1000
NC, NS, LANES = 2, 16, 16
NW = NC * NS
SB = 1024
NB_SC = SB // NW
PER_W = NB_SC * L
BB = 32
GROUPS = (0, 16, 32, 34)


@functools.cache
def _make_sc_gather():
    mesh = plsc.VectorSubcoreMesh(core_axis_name="c", subcore_axis_name="s")
    return functools.partial(
        pl.kernel,
        mesh=mesh,
        out_type=[
            jax.ShapeDtypeStruct((SB * L,), jnp.float32),
            jax.ShapeDtypeStruct((SB * L,), jnp.float32),
        ],
        scratch_types=[
            pltpu.VMEM((PER_W,), jnp.int32),
            pltpu.VMEM((PER_W,), jnp.int32),
            pltpu.VMEM((L, V), jnp.float32),
            pltpu.VMEM((L, V), jnp.float32),
            pltpu.VMEM((PER_W,), jnp.float32),
            pltpu.VMEM((PER_W,), jnp.float32),
            pltpu.SemaphoreType.DMA,
            pltpu.SemaphoreType.DMA,
        ],
        compiler_params=pltpu.CompilerParams(
            use_tc_tiling_on_sc=True,
            disable_bounds_checks=True,
        ),
    )(_sc_gather_body)


def _sc_gather_body(scores_hbm, pos_hbm, neg_hbm, pout_hbm, nout_hbm,
                    pidx_v, nidx_v, buf0, buf1, pg_v, ng_v, sem0, sem1):
    wid = lax.axis_index("s") * NC + lax.axis_index("c")
    base = wid * PER_W
    b0 = wid * NB_SC

    pltpu.sync_copy(pos_hbm.at[pl.ds(base, PER_W)], pidx_v)
    pltpu.sync_copy(neg_hbm.at[pl.ds(base, PER_W)], nidx_v)

    def _extract(buf, bi):
        off = bi * L
        lane = lax.iota(jnp.int32, LANES)
        for g in GROUPS:
            pi16 = pidx_v[pl.ds(off + g, LANES)]
            ni16 = nidx_v[pl.ds(off + g, LANES)]
            pc0 = pi16 & ~(LANES - 1)
            nc0 = ni16 & ~(LANES - 1)
            pcm = pi16 & (LANES - 1)
            ncm = ni16 & (LANES - 1)
            pacc = jnp.zeros((LANES,), jnp.float32)
            nacc = jnp.zeros((LANES,), jnp.float32)
            for k in range(LANES):
                row = g + k
                pv = buf[row, pl.ds(pl.multiple_of(pc0[k], LANES), LANES)]
                nv = buf[row, pl.ds(pl.multiple_of(nc0[k], LANES), LANES)]
                ps = jnp.take(pv, pcm)[k]
                ns = jnp.take(nv, ncm)[k]
                pacc = jnp.where(lane == k, ps, pacc)
                nacc = jnp.where(lane == k, ns, nacc)
            pg_v[pl.ds(off + g, LANES)] = pacc
            ng_v[pl.ds(off + g, LANES)] = nacc

    pltpu.make_async_copy(scores_hbm.at[b0], buf0, sem0).start()

    def _step(bi, carry):
        nxt = bi + 1

        @pl.when(nxt < NB_SC)
        def _prefetch():
            @pl.when(nxt % 2 == 0)
            def _():
                pltpu.make_async_copy(
                    scores_hbm.at[b0 + nxt], buf0, sem0).start()

            @pl.when(nxt % 2 == 1)
            def _():
                pltpu.make_async_copy(
                    scores_hbm.at[b0 + nxt], buf1, sem1).start()

        @pl.when(bi % 2 == 0)
        def _use0():
            pltpu.make_async_copy(scores_hbm.at[b0], buf0, sem0).wait()
            _extract(buf0, bi)

        @pl.when(bi % 2 == 1)
        def _use1():
            pltpu.make_async_copy(scores_hbm.at[b0], buf1, sem1).wait()
            _extract(buf1, bi)

        return carry

    lax.fori_loop(0, NB_SC, _step, 0)

    pltpu.sync_copy(pg_v, pout_hbm.at[pl.ds(base, PER_W)])
    pltpu.sync_copy(ng_v, nout_hbm.at[pl.ds(base, PER_W)])


def _tc_extract_body(s_ref, pn_ref, w_ref, acc_ref):
    i = pl.program_id(0)
    s = s_ref[...]
    pn = pn_ref[...]
    w = w_ref[...]
    hi = pn >> 7
    lo = pn & 127
    g = jnp.zeros(pn.shape, jnp.float32)
    for j in range(8):
        c0 = 128 * j
        cl = min(128, V - c0)
        idx = jnp.minimum(lo, cl - 1)
        gj = jnp.take_along_axis(
            s[:, :, c0:c0 + cl], idx, axis=2, mode="promise_in_bounds")
        g = g + jnp.where(hi == j, gj, 0.0)
    x = (g[..., 0] - g[..., 1]) * w
    ls = jnp.minimum(x, 0.0) - jnp.log1p(jnp.exp(-jnp.abs(x)))
    v = jnp.logical_and(pn[..., 0] > 0, pn[..., 1] > 0).astype(jnp.float32)

    @pl.when(i == 0)
    def _():
        acc_ref[...] = jnp.zeros((1, 2), jnp.float32)

    part = jnp.concatenate(
        [jnp.sum(ls * v).reshape(1, 1), jnp.sum(v).reshape(1, 1)], axis=1)
    acc_ref[...] += part


def _tc_extract(scores, pos, neg, w):
    nb = B - SB
    grid = (nb // BB,)
    pn = jnp.stack([pos, neg], axis=-1)
    off = SB // BB
    return pl.pallas_call(
        _tc_extract_body,
        grid=grid,
        in_specs=[
            pl.BlockSpec((BB, L, V), lambda i: (off + i, 0, 0)),
            pl.BlockSpec((BB, L, 2), lambda i: (off + i, 0, 0)),
            pl.BlockSpec((BB, L), lambda i: (off + i, 0)),
        ],
        out_specs=pl.BlockSpec((1, 2), lambda i: (0, 0)),
        out_shape=jax.ShapeDtypeStruct((1, 2), jnp.float32),
    )(scores, pn, w)


def _sc_loss_body(pg_ref, ng_ref, w_ref, pi_ref, ni_ref, out_ref):
    pg = pg_ref[...]
    ng = ng_ref[...]
    w = w_ref[...]
    valid = jnp.logical_and(pi_ref[...] > 0, ni_ref[...] > 0)
    x = (pg - ng) * w
    ls = jnp.minimum(x, 0.0) - jnp.log1p(jnp.exp(-jnp.abs(x)))
    v = valid.astype(jnp.float32)
    out_ref[...] = jnp.concatenate(
        [jnp.sum(ls * v).reshape(1, 1), jnp.sum(v).reshape(1, 1)], axis=1)


def kernel(scores, positive_mask, negative_mask, weights):
    pos = positive_mask.astype(jnp.int32)
    neg = negative_mask.astype(jnp.int32)
    acc_tc = _tc_extract(scores, pos, neg, weights) if SB < B else 0.0
    if SB > 0:
        nsc = SB * L
        psl = pos[:SB].reshape(-1)
        nsl = neg[:SB].reshape(-1)
        pg, ng = _make_sc_gather()(scores, psl, nsl)
        Rs, Cs = nsc // 128, 128
        acc_sc = pl.pallas_call(
            _sc_loss_body,
            out_shape=jax.ShapeDtypeStruct((1, 2), jnp.float32),
        )(pg.reshape(Rs, Cs), ng.reshape(Rs, Cs),
          weights[:SB].reshape(Rs, Cs),
          psl.reshape(Rs, Cs), nsl.reshape(Rs, Cs))
        acc = acc_tc + acc_sc
    else:
        acc = acc_tc
    s, c = acc[0, 0], acc[0, 1]
    return -jnp.where(c == 0.0, jnp.float32(0.0), s / jnp.maximum(c, 1.0))


# speedup vs baseline: 1.5249x; 1.0301x over previous
import functools

import jax
import jax.numpy as jnp
from jax import lax
from jax.experimental import pallas as pl
from jax.experimental.pallas import tpu as pltpu
from jax.experimental.pallas import tpu_sc as plsc

B, L, V = 1024, 50, 1000
NC, NS, LANES = 2, 16, 16
NW = NC * NS
SB = 1024
NB_SC = SB // NW
PER_W = NB_SC * L
BB = 32
GROUPS = (0, 16, 32, 34)


@functools.cache
def _make_sc_gather():
    mesh = plsc.VectorSubcoreMesh(core_axis_name="c", subcore_axis_name="s")
    return functools.partial(
        pl.kernel,
        mesh=mesh,
        out_type=[
            jax.ShapeDtypeStruct((SB * L,), jnp.float32),
            jax.ShapeDtypeStruct((SB * L,), jnp.float32),
        ],
        scratch_types=[
            pltpu.VMEM((PER_W,), jnp.int32),
            pltpu.VMEM((PER_W,), jnp.int32),
            pltpu.VMEM((L, V), jnp.float32),
            pltpu.VMEM((L, V), jnp.float32),
            pltpu.VMEM((PER_W,), jnp.float32),
            pltpu.VMEM((PER_W,), jnp.float32),
            pltpu.SemaphoreType.DMA,
            pltpu.SemaphoreType.DMA,
            pltpu.SemaphoreType.DMA,
            pltpu.SemaphoreType.DMA,
            pltpu.SemaphoreType.DMA,
            pltpu.SemaphoreType.DMA,
        ],
        compiler_params=pltpu.CompilerParams(
            use_tc_tiling_on_sc=True,
            disable_bounds_checks=True,
        ),
    )(_sc_gather_body)


def _sc_gather_body(scores_hbm, pos_hbm, neg_hbm, pout_hbm, nout_hbm,
                    pidx_v, nidx_v, buf0, buf1, pg_v, ng_v,
                    s0a, s0b, s0c, s1a, s1b, s1c):
    wid = lax.axis_index("s") * NC + lax.axis_index("c")
    base = wid * PER_W
    b0 = wid * NB_SC

    pltpu.sync_copy(pos_hbm.at[pl.ds(base, PER_W)], pidx_v)
    pltpu.sync_copy(neg_hbm.at[pl.ds(base, PER_W)], nidx_v)

    def _extract(buf, bi):
        off = bi * L
        lane = lax.iota(jnp.int32, LANES)
        for g in GROUPS:
            pi16 = pidx_v[pl.ds(off + g, LANES)]
            ni16 = nidx_v[pl.ds(off + g, LANES)]
            pc0 = pi16 & ~(LANES - 1)
            nc0 = ni16 & ~(LANES - 1)
            pcm = pi16 & (LANES - 1)
            ncm = ni16 & (LANES - 1)
            pacc = jnp.zeros((LANES,), jnp.float32)
            nacc = jnp.zeros((LANES,), jnp.float32)
            for k in range(LANES):
                row = g + k
                pv = buf[row, pl.ds(pl.multiple_of(pc0[k], LANES), LANES)]
                nv = buf[row, pl.ds(pl.multiple_of(nc0[k], LANES), LANES)]
                ps = jnp.take(pv, pcm)[k]
                ns = jnp.take(nv, ncm)[k]
                pacc = jnp.where(lane == k, ps, pacc)
                nacc = jnp.where(lane == k, ns, nacc)
            pg_v[pl.ds(off + g, LANES)] = pacc
            ng_v[pl.ds(off + g, LANES)] = nacc

    RC = ((0, 16), (16, 16), (32, 18))

    def _fill(buf, b, sems):
        for (r0, rn), sem in zip(RC, sems):
            pltpu.make_async_copy(scores_hbm.at[b, pl.ds(r0, rn)],
                                  buf.at[pl.ds(r0, rn)], sem).start()

    def _drain(buf, sems):
        for (r0, rn), sem in zip(RC, sems):
            pltpu.make_async_copy(scores_hbm.at[b0, pl.ds(r0, rn)],
                                  buf.at[pl.ds(r0, rn)], sem).wait()

    _fill(buf0, b0, (s0a, s0b, s0c))

    def _step(bi, carry):
        nxt = bi + 1

        @pl.when(nxt < NB_SC)
        def _prefetch():
            @pl.when(nxt % 2 == 0)
            def _():
                _fill(buf0, b0 + nxt, (s0a, s0b, s0c))

            @pl.when(nxt % 2 == 1)
            def _():
                _fill(buf1, b0 + nxt, (s1a, s1b, s1c))

        @pl.when(bi % 2 == 0)
        def _use0():
            _drain(buf0, (s0a, s0b, s0c))
            _extract(buf0, bi)

        @pl.when(bi % 2 == 1)
        def _use1():
            _drain(buf1, (s1a, s1b, s1c))
            _extract(buf1, bi)

        return carry

    lax.fori_loop(0, NB_SC, _step, 0)

    pltpu.sync_copy(pg_v, pout_hbm.at[pl.ds(base, PER_W)])
    pltpu.sync_copy(ng_v, nout_hbm.at[pl.ds(base, PER_W)])


def _tc_extract_body(s_ref, pn_ref, w_ref, acc_ref):
    i = pl.program_id(0)
    s = s_ref[...]
    pn = pn_ref[...]
    w = w_ref[...]
    hi = pn >> 7
    lo = pn & 127
    g = jnp.zeros(pn.shape, jnp.float32)
    for j in range(8):
        c0 = 128 * j
        cl = min(128, V - c0)
        idx = jnp.minimum(lo, cl - 1)
        gj = jnp.take_along_axis(
            s[:, :, c0:c0 + cl], idx, axis=2, mode="promise_in_bounds")
        g = g + jnp.where(hi == j, gj, 0.0)
    x = (g[..., 0] - g[..., 1]) * w
    ls = jnp.minimum(x, 0.0) - jnp.log1p(jnp.exp(-jnp.abs(x)))
    v = jnp.logical_and(pn[..., 0] > 0, pn[..., 1] > 0).astype(jnp.float32)

    @pl.when(i == 0)
    def _():
        acc_ref[...] = jnp.zeros((1, 2), jnp.float32)

    part = jnp.concatenate(
        [jnp.sum(ls * v).reshape(1, 1), jnp.sum(v).reshape(1, 1)], axis=1)
    acc_ref[...] += part


def _tc_extract(scores, pos, neg, w):
    nb = B - SB
    grid = (nb // BB,)
    pn = jnp.stack([pos, neg], axis=-1)
    off = SB // BB
    return pl.pallas_call(
        _tc_extract_body,
        grid=grid,
        in_specs=[
            pl.BlockSpec((BB, L, V), lambda i: (off + i, 0, 0)),
            pl.BlockSpec((BB, L, 2), lambda i: (off + i, 0, 0)),
            pl.BlockSpec((BB, L), lambda i: (off + i, 0)),
        ],
        out_specs=pl.BlockSpec((1, 2), lambda i: (0, 0)),
        out_shape=jax.ShapeDtypeStruct((1, 2), jnp.float32),
    )(scores, pn, w)


def _sc_loss_body(pg_ref, ng_ref, w_ref, pi_ref, ni_ref, out_ref):
    pg = pg_ref[...]
    ng = ng_ref[...]
    w = w_ref[...]
    valid = jnp.logical_and(pi_ref[...] > 0, ni_ref[...] > 0)
    x = (pg - ng) * w
    ls = jnp.minimum(x, 0.0) - jnp.log1p(jnp.exp(-jnp.abs(x)))
    v = valid.astype(jnp.float32)
    out_ref[...] = jnp.concatenate(
        [jnp.sum(ls * v).reshape(1, 1), jnp.sum(v).reshape(1, 1)], axis=1)


def kernel(scores, positive_mask, negative_mask, weights):
    pos = positive_mask.astype(jnp.int32)
    neg = negative_mask.astype(jnp.int32)
    acc_tc = _tc_extract(scores, pos, neg, weights) if SB < B else 0.0
    if SB > 0:
        nsc = SB * L
        psl = pos[:SB].reshape(-1)
        nsl = neg[:SB].reshape(-1)
        pg, ng = _make_sc_gather()(scores, psl, nsl)
        Rs, Cs = nsc // 128, 128
        acc_sc = pl.pallas_call(
            _sc_loss_body,
            out_shape=jax.ShapeDtypeStruct((1, 2), jnp.float32),
        )(pg.reshape(Rs, Cs), ng.reshape(Rs, Cs),
          weights[:SB].reshape(Rs, Cs),
          psl.reshape(Rs, Cs), nsl.reshape(Rs, Cs))
        acc = acc_tc + acc_sc
    else:
        acc = acc_tc
    s, c = acc[0, 0], acc[0, 1]
    return -jnp.where(c == 0.0, jnp.float32(0.0), s / jnp.maximum(c, 1.0))
